# baseline (speedup 1.0000x reference)
import functools

import jax
import jax.numpy as jnp
from jax import lax
from jax.experimental import pallas as pl
from jax.experimental.pallas import tpu as pltpu
from jax.experimental.pallas import tpu_sc as plsc

NUM_NODES = 1000000
EMBED_DIM = 64
BATCH = 16384

_info = plsc.get_sparse_core_info()
_NC, _NS = _info.num_cores, _info.num_subcores
_NW = _NC * _NS
_B_PER_W = BATCH // _NW
_BLK = 128

_mesh = plsc.VectorSubcoreMesh(core_axis_name="c", subcore_axis_name="s")


@functools.partial(
    pl.kernel,
    mesh=_mesh,
    out_type=jax.ShapeDtypeStruct((BATCH, EMBED_DIM), jnp.float32),
    scratch_types=[
        pltpu.VMEM((_B_PER_W,), jnp.int32),
        pltpu.VMEM((EMBED_DIM, _BLK), jnp.float32),
        pltpu.VMEM((EMBED_DIM, _BLK), jnp.float32),
        pltpu.VMEM((_B_PER_W, EMBED_DIM), jnp.float32),
        pltpu.SemaphoreType.DMA,
        pltpu.SemaphoreType.DMA,
    ],
    compiler_params=pltpu.CompilerParams(needs_layout_passes=False),
)
def _gather_kernel(idx_hbm, tableT_hbm, out_hbm, idx_v, t0, t1, rows_v,
                   sem0, sem1):
    wid = lax.axis_index("s") * _NC + lax.axis_index("c")
    base = wid * _B_PER_W
    pltpu.sync_copy(idx_hbm.at[pl.ds(base, _B_PER_W)], idx_v)

    tbufs = (t0, t1)
    sems = (sem0, sem1)
    c16 = [lax.iota(jnp.int32, 16) + 16 * k for k in range(EMBED_DIM // 16)]

    def start_fetch(r, b):
        rbase = pl.multiple_of(r & ~(_BLK - 1), _BLK)
        pltpu.async_copy(tableT_hbm.at[:, pl.ds(rbase, _BLK)],
                         tbufs[b], sems[b])

    def wait_fetch(b):
        pltpu.make_async_copy(tableT_hbm.at[:, pl.ds(0, _BLK)],
                              tbufs[b], sems[b]).wait()

    def extract(j, r, b):
        rr = (c16[0] & 0) + (r & (_BLK - 1))
        for k in range(EMBED_DIM // 16):
            flat = tbufs[b].reshape(1, EMBED_DIM * _BLK).at[0]
            vals = plsc.load_gather(flat, [c16[k] * _BLK + rr])
            rows_v[j, pl.ds(16 * k, 16)] = vals

    v0 = idx_v[pl.ds(0, 16)]
    start_fetch(v0[0], 0)

    def body(g, carry):
        v = idx_v[pl.ds(16 * g, 16)]
        for u in range(16):
            r = v[u]
            b = u & 1
            wait_fetch(b)
            if u < 15:
                start_fetch(v[u + 1], b ^ 1)
            else:
                @pl.when(g + 1 < _B_PER_W // 16)
                def _():
                    vn = idx_v[pl.ds(16 * (g + 1), 16)]
                    start_fetch(vn[0], 0)
            extract(16 * g + u, r, b)
        return carry

    lax.fori_loop(0, _B_PER_W // 16, body, 0)

    pltpu.sync_copy(rows_v, out_hbm.at[pl.ds(base, _B_PER_W)])


def kernel(nodes, ent_features):
    return _gather_kernel(nodes.astype(jnp.int32), ent_features.T)


# speedup vs baseline: 1.9679x; 1.9679x over previous
import functools

import jax
import jax.numpy as jnp
from jax import lax
from jax.experimental import pallas as pl
from jax.experimental.pallas import tpu as pltpu
from jax.experimental.pallas import tpu_sc as plsc

NUM_NODES = 1000000
EMBED_DIM = 64
BATCH = 16384

_info = plsc.get_sparse_core_info()
_NC, _NS = _info.num_cores, _info.num_subcores
_NW = _NC * _NS
_B_PER_W = BATCH // _NW
_BLK = 128

_mesh = plsc.VectorSubcoreMesh(core_axis_name="c", subcore_axis_name="s")


@functools.partial(
    pl.kernel,
    mesh=_mesh,
    out_type=jax.ShapeDtypeStruct((BATCH, EMBED_DIM), jnp.float32),
    scratch_types=[
        pltpu.VMEM((_B_PER_W,), jnp.int32),
        pltpu.VMEM((EMBED_DIM, _BLK), jnp.float32),
        pltpu.VMEM((EMBED_DIM, _BLK), jnp.float32),
        pltpu.VMEM((EMBED_DIM, _BLK), jnp.float32),
        pltpu.VMEM((EMBED_DIM, _BLK), jnp.float32),
        pltpu.VMEM((_B_PER_W, EMBED_DIM), jnp.float32),
        pltpu.SemaphoreType.DMA,
        pltpu.SemaphoreType.DMA,
        pltpu.SemaphoreType.DMA,
        pltpu.SemaphoreType.DMA,
    ],
    compiler_params=pltpu.CompilerParams(needs_layout_passes=False),
)
def _gather_kernel(idx_hbm, tableT_hbm, out_hbm, idx_v, t0, t1, t2, t3,
                   rows_v, sem0, sem1, sem2, sem3):
    wid = lax.axis_index("s") * _NC + lax.axis_index("c")
    base = wid * _B_PER_W
    pltpu.sync_copy(idx_hbm.at[pl.ds(base, _B_PER_W)], idx_v)

    tbufs = (t0, t1, t2, t3)
    sems = (sem0, sem1, sem2, sem3)
    c16 = [lax.iota(jnp.int32, 16) + 16 * k for k in range(EMBED_DIM // 16)]

    def start_fetch(r, b):
        rbase = pl.multiple_of(r & ~(_BLK - 1), _BLK)
        pltpu.async_copy(tableT_hbm.at[:, pl.ds(rbase, _BLK)],
                         tbufs[b], sems[b])

    def wait_fetch(b):
        pltpu.make_async_copy(tableT_hbm.at[:, pl.ds(0, _BLK)],
                              tbufs[b], sems[b]).wait()

    def extract(j, r, b):
        rr = (c16[0] & 0) + (r & (_BLK - 1))
        for k in range(EMBED_DIM // 16):
            flat = tbufs[b].reshape(1, EMBED_DIM * _BLK).at[0]
            vals = plsc.load_gather(flat, [c16[k] * _BLK + rr])
            rows_v[j, pl.ds(16 * k, 16)] = vals

    v0 = idx_v[pl.ds(0, 16)]
    for d in range(3):
        start_fetch(v0[d], d)

    def body(g, carry):
        v = idx_v[pl.ds(16 * g, 16)]
        for u in range(16):
            r = v[u]
            b = u & 3
            wait_fetch(b)
            nb = (u + 3) & 3
            if u < 13:
                start_fetch(v[u + 3], nb)
            else:
                @pl.when(g + 1 < _B_PER_W // 16)
                def _():
                    vn = idx_v[pl.ds(16 * (g + 1), 16)]
                    start_fetch(vn[u - 13], nb)
            extract(16 * g + u, r, b)
        return carry

    lax.fori_loop(0, _B_PER_W // 16, body, 0)

    pltpu.sync_copy(rows_v, out_hbm.at[pl.ds(base, _B_PER_W)])


def kernel(nodes, ent_features):
    return _gather_kernel(nodes.astype(jnp.int32), ent_features.T)


# speedup vs baseline: 2.4855x; 1.2630x over previous
import functools

import jax
import jax.numpy as jnp
from jax import lax
from jax.experimental import pallas as pl
from jax.experimental.pallas import tpu as pltpu
from jax.experimental.pallas import tpu_sc as plsc

NUM_NODES = 1000000
EMBED_DIM = 64
BATCH = 16384

_info = plsc.get_sparse_core_info()
_NC, _NS = _info.num_cores, _info.num_subcores
_NW = _NC * _NS
_B_PER_W = BATCH // _NW
_BLK = 128

_mesh = plsc.VectorSubcoreMesh(core_axis_name="c", subcore_axis_name="s")


@functools.partial(
    pl.kernel,
    mesh=_mesh,
    out_type=jax.ShapeDtypeStruct((BATCH, EMBED_DIM), jnp.float32),
    scratch_types=[
        pltpu.VMEM((_B_PER_W,), jnp.int32),
        *[pltpu.VMEM((EMBED_DIM, _BLK), jnp.float32) for _ in range(8)],
        pltpu.VMEM((_B_PER_W // 2, EMBED_DIM), jnp.float32),
        *[pltpu.SemaphoreType.DMA for _ in range(8)],
        pltpu.SemaphoreType.DMA,
    ],
    compiler_params=pltpu.CompilerParams(needs_layout_passes=False),
)
def _gather_kernel(idx_hbm, tableT_hbm, out_hbm, idx_v,
                   t0, t1, t2, t3, t4, t5, t6, t7, rows_v,
                   sem0, sem1, sem2, sem3, sem4, sem5, sem6, sem7, semw):
    wid = lax.axis_index("s") * _NC + lax.axis_index("c")
    base = wid * _B_PER_W
    pltpu.sync_copy(idx_hbm.at[pl.ds(base, _B_PER_W)], idx_v)

    tbufs = (t0, t1, t2, t3, t4, t5, t6, t7)
    sems = (sem0, sem1, sem2, sem3, sem4, sem5, sem6, sem7)
    c16 = [lax.iota(jnp.int32, 16) + 16 * k for k in range(EMBED_DIM // 16)]

    def start_fetch(r, b):
        rbase = pl.multiple_of(r & ~(_BLK - 1), _BLK)
        pltpu.async_copy(tableT_hbm.at[:, pl.ds(rbase, _BLK)],
                         tbufs[b], sems[b])

    def wait_fetch(b):
        pltpu.make_async_copy(tableT_hbm.at[:, pl.ds(0, _BLK)],
                              tbufs[b], sems[b]).wait()

    def extract(j, r, b):
        rr = (c16[0] & 0) + (r & (_BLK - 1))
        for k in range(EMBED_DIM // 16):
            flat = tbufs[b].reshape(1, EMBED_DIM * _BLK).at[0]
            vals = plsc.load_gather(flat, [c16[k] * _BLK + rr])
            rows_v[j, pl.ds(16 * k, 16)] = vals

    _DEPTH = 7
    _NG = _B_PER_W // 16
    _HG = _NG // 2

    v0 = idx_v[pl.ds(0, 16)]
    for d in range(_DEPTH):
        start_fetch(v0[d], d)

    def make_body(half):
        def body(g, carry):
            j0 = 16 * (half * _HG + g)
            v = idx_v[pl.ds(j0, 16)]
            for u in range(16):
                r = v[u]
                b = u & 7
                wait_fetch(b)
                nb = (u + _DEPTH) & 7
                if u < 16 - _DEPTH:
                    start_fetch(v[u + _DEPTH], nb)
                else:
                    @pl.when(j0 + 16 + (u - (16 - _DEPTH)) < _B_PER_W)
                    def _():
                        vn = idx_v[pl.ds(j0 + 16, 16)]
                        start_fetch(vn[u - (16 - _DEPTH)], nb)
                extract(16 * g + u, r, b)
            return carry
        return body

    lax.fori_loop(0, _HG, make_body(0), 0)
    pltpu.sync_copy(rows_v, out_hbm.at[pl.ds(base, _B_PER_W // 2)])
    lax.fori_loop(0, _HG, make_body(1), 0)
    pltpu.sync_copy(rows_v,
                    out_hbm.at[pl.ds(base + _B_PER_W // 2, _B_PER_W // 2)])


def kernel(nodes, ent_features):
    return _gather_kernel(nodes.astype(jnp.int32), ent_features.T)
